# baseline (speedup 1.0000x reference)
import functools

import jax
import jax.numpy as jnp
from jax import lax
from jax.experimental import pallas as pl
from jax.experimental.pallas import tpu as pltpu
from jax.experimental.pallas import tpu_sc as plsc

_N = 10000
_E = 320000
_IN = 128
_HID = 16
_HEADS = 8
_OUT = 64

_NP = 10240
_NW = 32
_ET = _E + _N
_EPWU = 10496
_EPWA = _EPWU + 256
_ETP = _EPWA * _NW
_RPT = _NP // 16


def _sc_edge_kernel(F, HEADS, C):
    NCH = _EPWU // C
    mesh = plsc.VectorSubcoreMesh(core_axis_name="c", subcore_axis_name="s")
    out_type = [
        jax.ShapeDtypeStruct((2, _NP, F), jnp.float32),
        jax.ShapeDtypeStruct((2, _NP, 16), jnp.float32),
    ]
    scratch = [
        pltpu.VMEM((2, C), jnp.int32),
        pltpu.VMEM((2, C), jnp.int32),
        pltpu.VMEM((2, C, 16), jnp.float32),
        pltpu.VMEM((2, C, 16), jnp.float32),
        pltpu.VMEM((2, C, 16), jnp.float32),
        pltpu.VMEM((2, C, F), jnp.float32),
        pltpu.VMEM((16,), jnp.float32),
        pltpu.VMEM_SHARED((_NP, F), jnp.float32),
        pltpu.VMEM_SHARED((_NP, 16), jnp.float32),
        pltpu.SemaphoreType.DMA,
        pltpu.SemaphoreType.DMA,
        pltpu.SemaphoreType.DMA,
        pltpu.SemaphoreType.DMA,
        pltpu.SemaphoreType.DMA,
    ]

    @functools.partial(
        pl.kernel, mesh=mesh, out_type=out_type, scratch_types=scratch,
        compiler_params=pltpu.CompilerParams(use_tc_tiling_on_sc=False))
    def k(srcR, dstR, hT, asT, adT, amx, zN, zD, numP, denP,
          idxS, idxD, asg, adg, exv, hg, amv, accN, accD,
          isem0, isem1, gsem0, gsem1, ssem):
        c = lax.axis_index("c")
        s = lax.axis_index("s")
        wid = s * 2 + c
        isem = [isem0, isem1]
        gsem = [gsem0, gsem1]

        @pl.when(s == 0)
        def _init():
            pltpu.sync_copy(zN, accN)
            pltpu.sync_copy(zD, accD)

        pltpu.sync_copy(amx, amv)
        plsc.subcore_barrier()
        amxv = amv[...]

        def idx_dma(g, b):
            base = wid * _EPWA + g * C
            return (pltpu.make_async_copy(srcR.at[pl.ds(base, C)],
                                          idxS.at[b], isem[b]),
                    pltpu.make_async_copy(dstR.at[pl.ds(base, C)],
                                          idxD.at[b], isem[b]))

        def gather_dma(b):
            return (pltpu.make_async_copy(asT.at[idxS.at[b]], asg.at[b],
                                          gsem[b]),
                    pltpu.make_async_copy(adT.at[idxD.at[b]], adg.at[b],
                                          gsem[b]),
                    pltpu.make_async_copy(hT.at[idxS.at[b]], hg.at[b],
                                          gsem[b]))

        def compute(b):
            def edge_body(e, cc):
                vs = asg[b, e, :]
                vd = adg[b, e, :]
                z = vs + vd
                ee = jnp.maximum(z, 0.2 * z)
                zm = amxv + vd
                mub = jnp.maximum(zm, 0.2 * zm)
                ex = jnp.exp(ee - mub)
                exv[b, e, :] = ex
                if HEADS == 1:
                    for q in range(F // 16):
                        hg[b, e, pl.ds(q * 16, 16)] = (
                            hg[b, e, pl.ds(q * 16, 16)] * ex)
                else:
                    for h in range(HEADS):
                        bb = lax.gather(
                            ex, jnp.full((16, 1), h, jnp.int32),
                            lax.GatherDimensionNumbers(
                                offset_dims=(), collapsed_slice_dims=(0,),
                                start_index_map=(0,)),
                            (1,),
                            mode=lax.GatherScatterMode.PROMISE_IN_BOUNDS)
                        hg[b, e, pl.ds(h * 16, 16)] = (
                            hg[b, e, pl.ds(h * 16, 16)] * bb)
                return cc

            lax.fori_loop(0, C, edge_body, 0, unroll=4)

        for d in idx_dma(0, 0):
            d.start()
        for d in idx_dma(1, 1):
            d.start()
        for d in idx_dma(0, 0):
            d.wait()
        for d in gather_dma(0):
            d.start()

        def loop_body(t, carry):
            for b in range(2):
                g = 2 * t + b
                nb = 1 - b
                for d in idx_dma(g + 1, nb):
                    d.wait()
                for d in gather_dma(nb):
                    d.start()
                for d in gather_dma(b):
                    d.wait()
                compute(b)
                d1 = pltpu.async_copy(exv.at[b], accD.at[idxD.at[b]],
                                      ssem, add=True)
                d2 = pltpu.async_copy(hg.at[b], accN.at[idxD.at[b]],
                                      ssem, add=True)
                d1.wait()
                d2.wait()
                for d in idx_dma(g + 2, b):
                    d.start()
            return carry

        lax.fori_loop(0, NCH // 2, loop_body, 0)

        for d in gather_dma(0):
            d.wait()
        for d in idx_dma(NCH + 1, 1):
            d.wait()

        plsc.subcore_barrier()
        r0 = s * _RPT
        pltpu.sync_copy(accN.at[pl.ds(r0, _RPT)],
                        numP.at[c, pl.ds(r0, _RPT)])
        pltpu.sync_copy(accD.at[pl.ds(r0, _RPT)],
                        denP.at[c, pl.ds(r0, _RPT)])

    return k


def _tc1(x, W1, As1, Ad1):
    def body(x_ref, w_ref, as_ref, ad_ref, h_ref, ast_ref, adt_ref, amx_ref):
        h = jnp.dot(x_ref[...], w_ref[...], preferred_element_type=jnp.float32)
        h_ref[...] = h
        a_s = jnp.dot(h, as_ref[...], preferred_element_type=jnp.float32)
        a_d = jnp.dot(h, ad_ref[...], preferred_element_type=jnp.float32)
        ast_ref[...] = jnp.concatenate([a_s, a_s], axis=1)
        adt_ref[...] = jnp.concatenate([a_d, a_d], axis=1)
        m = jnp.max(a_s, axis=0, keepdims=True)
        amx_ref[...] = jnp.concatenate([m, m], axis=1)

    return pl.pallas_call(
        body,
        out_shape=[
            jax.ShapeDtypeStruct((_N, _IN), jnp.float32),
            jax.ShapeDtypeStruct((_N, 16), jnp.float32),
            jax.ShapeDtypeStruct((_N, 16), jnp.float32),
            jax.ShapeDtypeStruct((1, 16), jnp.float32),
        ],
    )(x, W1, As1, Ad1)


def _tc2(numP, denP, b1, W2, a_src2, a_dst2, EXP8, ONES16):
    def body(n_ref, d_ref, b1_ref, w2_ref, as2_ref, ad2_ref, e8_ref,
             o16_ref, h2_ref, ast_ref, adt_ref, amx_ref):
        num = n_ref[0, : _N, :] + n_ref[1, : _N, :]
        den = d_ref[0, : _N, 0:8] + d_ref[1, : _N, 0:8]
        rec = 1.0 / (den + 1e-16)
        recb = jnp.dot(rec, e8_ref[...], preferred_element_type=jnp.float32)
        hin = num * recb + b1_ref[...]
        hin = jnp.where(hin > 0, hin, jnp.exp(jnp.minimum(hin, 0.0)) - 1.0)
        h2 = jnp.dot(hin, w2_ref[...], preferred_element_type=jnp.float32)
        h2_ref[...] = h2
        a_s = jnp.dot(h2, as2_ref[...], preferred_element_type=jnp.float32)
        a_d = jnp.dot(h2, ad2_ref[...], preferred_element_type=jnp.float32)
        ast_ref[...] = jnp.dot(a_s, o16_ref[...],
                               preferred_element_type=jnp.float32)
        adt_ref[...] = jnp.dot(a_d, o16_ref[...],
                               preferred_element_type=jnp.float32)
        m = jnp.max(a_s, axis=0, keepdims=True)
        amx_ref[...] = jnp.dot(m, o16_ref[...],
                               preferred_element_type=jnp.float32)

    return pl.pallas_call(
        body,
        out_shape=[
            jax.ShapeDtypeStruct((_N, _OUT), jnp.float32),
            jax.ShapeDtypeStruct((_N, 16), jnp.float32),
            jax.ShapeDtypeStruct((_N, 16), jnp.float32),
            jax.ShapeDtypeStruct((1, 16), jnp.float32),
        ],
    )(numP, denP, b1.reshape(1, -1), W2, a_src2.T, a_dst2.T, EXP8, ONES16)


def _tc3(numP, denP, b2, ONES64):
    def body(n_ref, d_ref, b2_ref, o64_ref, out_ref):
        num = n_ref[0, : _N, :] + n_ref[1, : _N, :]
        den = d_ref[0, : _N, 0:1] + d_ref[1, : _N, 0:1]
        rec = 1.0 / (den + 1e-16)
        recb = jnp.dot(rec, o64_ref[...], preferred_element_type=jnp.float32)
        out_ref[...] = num * recb + b2_ref[...]

    return pl.pallas_call(
        body,
        out_shape=jax.ShapeDtypeStruct((_N, _OUT), jnp.float32),
    )(numP, denP, b2.reshape(1, -1), ONES64)


def kernel(x, edge_index, W1, a_src1, a_dst1, b1, W2, a_src2, a_dst2, b2):
    loop = jnp.arange(_N, dtype=jnp.int32)
    pad = _NW * _EPWU - _ET
    src_u = jnp.concatenate(
        [edge_index[0], loop, jnp.zeros((pad,), jnp.int32)])
    dst_u = jnp.concatenate(
        [edge_index[1], loop, jnp.full((pad,), _N, jnp.int32)])
    slack = jnp.zeros((_NW, 256), jnp.int32)
    srcp = jnp.concatenate(
        [src_u.reshape(_NW, _EPWU), slack], axis=1).reshape(-1)
    dstp = jnp.concatenate(
        [dst_u.reshape(_NW, _EPWU), slack], axis=1).reshape(-1)

    eye8 = jnp.eye(_HEADS, dtype=jnp.float32)
    As1 = (a_src1[:, :, None] * eye8[:, None, :]).reshape(_IN, _HEADS)
    Ad1 = (a_dst1[:, :, None] * eye8[:, None, :]).reshape(_IN, _HEADS)
    EXP8 = jnp.repeat(eye8, _HID, axis=1)
    ONES16 = jnp.ones((1, 16), jnp.float32)
    ONES64 = jnp.ones((1, _OUT), jnp.float32)

    zN1 = jnp.zeros((_NP, _IN), jnp.float32)
    zN2 = jnp.zeros((_NP, _OUT), jnp.float32)
    zD = jnp.zeros((_NP, 16), jnp.float32)

    tpad = jnp.zeros((_NP - _N, 16), jnp.float32)

    h1, asT1, adT1, amx1 = _tc1(x, W1, As1, Ad1)
    num1, den1 = _sc_edge_kernel(_IN, _HEADS, 64)(
        srcp, dstp, h1, jnp.concatenate([asT1, tpad]),
        jnp.concatenate([adT1, tpad]), amx1.reshape(16), zN1, zD)

    h2, asT2, adT2, amx2 = _tc2(num1, den1, b1, W2, a_src2, a_dst2,
                                EXP8, ONES16)
    num2, den2 = _sc_edge_kernel(_OUT, 1, 128)(
        srcp, dstp, h2, jnp.concatenate([asT2, tpad]),
        jnp.concatenate([adT2, tpad]), amx2.reshape(16), zN2, zD)

    return _tc3(num2, den2, b2, ONES64)


# speedup vs baseline: 1.1596x; 1.1596x over previous
import functools

import jax
import jax.numpy as jnp
from jax import lax
from jax.experimental import pallas as pl
from jax.experimental.pallas import tpu as pltpu
from jax.experimental.pallas import tpu_sc as plsc

_N = 10000
_E = 320000
_IN = 128
_HID = 16
_HEADS = 8
_OUT = 64

_NP = 10240
_NW = 32
_ET = _E + _N
_EPWU = 10496
_EPWA = _EPWU + 256
_ETP = _EPWA * _NW
_RPT = _NP // 16


def _sc_edge_kernel(F, HEADS, C):
    NCH = _EPWU // C
    mesh = plsc.VectorSubcoreMesh(core_axis_name="c", subcore_axis_name="s")
    out_type = [
        jax.ShapeDtypeStruct((2, _NP, F), jnp.float32),
        jax.ShapeDtypeStruct((2, _NP, 16), jnp.float32),
    ]
    scratch = [
        pltpu.VMEM((2, C), jnp.int32),
        pltpu.VMEM((2, C), jnp.int32),
        pltpu.VMEM((2, C, 16), jnp.float32),
        pltpu.VMEM((2, C, 16), jnp.float32),
        pltpu.VMEM((2, C, 16), jnp.float32),
        pltpu.VMEM((2, C, F), jnp.float32),
        pltpu.VMEM((16,), jnp.float32),
        pltpu.VMEM_SHARED((_NP, F), jnp.float32),
        pltpu.VMEM_SHARED((_NP, 16), jnp.float32),
        pltpu.SemaphoreType.DMA,
        pltpu.SemaphoreType.DMA,
        pltpu.SemaphoreType.DMA,
        pltpu.SemaphoreType.DMA,
        pltpu.SemaphoreType.DMA,
    ]

    @functools.partial(
        pl.kernel, mesh=mesh, out_type=out_type, scratch_types=scratch,
        compiler_params=pltpu.CompilerParams(use_tc_tiling_on_sc=False))
    def k(srcR, dstR, hT, asT, adT, amx, zN, zD, numP, denP,
          idxS, idxD, asg, adg, exv, hg, amv, accN, accD,
          isem0, isem1, gsem0, gsem1, ssem):
        c = lax.axis_index("c")
        s = lax.axis_index("s")
        wid = s * 2 + c
        isem = [isem0, isem1]
        gsem = [gsem0, gsem1]

        @pl.when(s == 0)
        def _init():
            pltpu.sync_copy(zN, accN)
            pltpu.sync_copy(zD, accD)

        pltpu.sync_copy(amx, amv)
        plsc.subcore_barrier()
        amxv = amv[...]

        def idx_dma(g, b):
            base = wid * _EPWA + g * C
            return (pltpu.make_async_copy(srcR.at[pl.ds(base, C)],
                                          idxS.at[b], isem[b]),
                    pltpu.make_async_copy(dstR.at[pl.ds(base, C)],
                                          idxD.at[b], isem[b]))

        def gather_dma(b):
            return (pltpu.make_async_copy(asT.at[idxS.at[b]], asg.at[b],
                                          gsem[b]),
                    pltpu.make_async_copy(adT.at[idxD.at[b]], adg.at[b],
                                          gsem[b]),
                    pltpu.make_async_copy(hT.at[idxS.at[b]], hg.at[b],
                                          gsem[b]))

        def compute(b):
            @plsc.parallel_loop(0, C, 1, unroll=4)
            def edge_body(e):
                vs = asg[b, e, :]
                vd = adg[b, e, :]
                z = vs + vd
                ee = jnp.maximum(z, 0.2 * z)
                zm = amxv + vd
                mub = jnp.maximum(zm, 0.2 * zm)
                ex = jnp.exp(ee - mub)
                exv[b, e, :] = ex
                if HEADS == 1:
                    for q in range(F // 16):
                        hg[b, e, pl.ds(q * 16, 16)] = (
                            hg[b, e, pl.ds(q * 16, 16)] * ex)
                else:
                    for h in range(HEADS):
                        bb = lax.gather(
                            ex, jnp.full((16, 1), h, jnp.int32),
                            lax.GatherDimensionNumbers(
                                offset_dims=(), collapsed_slice_dims=(0,),
                                start_index_map=(0,)),
                            (1,),
                            mode=lax.GatherScatterMode.PROMISE_IN_BOUNDS)
                        hg[b, e, pl.ds(h * 16, 16)] = (
                            hg[b, e, pl.ds(h * 16, 16)] * bb)

        for d in idx_dma(0, 0):
            d.start()
        for d in idx_dma(1, 1):
            d.start()
        for d in idx_dma(0, 0):
            d.wait()
        for d in gather_dma(0):
            d.start()

        def loop_body(t, carry):
            for b in range(2):
                g = 2 * t + b
                nb = 1 - b
                for d in idx_dma(g + 1, nb):
                    d.wait()
                for d in gather_dma(nb):
                    d.start()
                for d in gather_dma(b):
                    d.wait()
                compute(b)
                d1 = pltpu.async_copy(exv.at[b], accD.at[idxD.at[b]],
                                      ssem, add=True)
                d2 = pltpu.async_copy(hg.at[b], accN.at[idxD.at[b]],
                                      ssem, add=True)
                d1.wait()
                d2.wait()
                for d in idx_dma(g + 2, b):
                    d.start()
            return carry

        lax.fori_loop(0, NCH // 2, loop_body, 0)

        for d in gather_dma(0):
            d.wait()
        for d in idx_dma(NCH + 1, 1):
            d.wait()

        plsc.subcore_barrier()
        r0 = s * _RPT
        pltpu.sync_copy(accN.at[pl.ds(r0, _RPT)],
                        numP.at[c, pl.ds(r0, _RPT)])
        pltpu.sync_copy(accD.at[pl.ds(r0, _RPT)],
                        denP.at[c, pl.ds(r0, _RPT)])

    return k


def _tc1(x, W1, As1, Ad1):
    def body(x_ref, w_ref, as_ref, ad_ref, h_ref, ast_ref, adt_ref, amx_ref):
        h = jnp.dot(x_ref[...], w_ref[...], preferred_element_type=jnp.float32)
        h_ref[...] = h
        a_s = jnp.dot(h, as_ref[...], preferred_element_type=jnp.float32)
        a_d = jnp.dot(h, ad_ref[...], preferred_element_type=jnp.float32)
        ast_ref[...] = jnp.concatenate([a_s, a_s], axis=1)
        adt_ref[...] = jnp.concatenate([a_d, a_d], axis=1)
        m = jnp.max(a_s, axis=0, keepdims=True)
        amx_ref[...] = jnp.concatenate([m, m], axis=1)

    return pl.pallas_call(
        body,
        out_shape=[
            jax.ShapeDtypeStruct((_N, _IN), jnp.float32),
            jax.ShapeDtypeStruct((_N, 16), jnp.float32),
            jax.ShapeDtypeStruct((_N, 16), jnp.float32),
            jax.ShapeDtypeStruct((1, 16), jnp.float32),
        ],
    )(x, W1, As1, Ad1)


def _tc2(numP, denP, b1, W2, a_src2, a_dst2, EXP8, ONES16):
    def body(n_ref, d_ref, b1_ref, w2_ref, as2_ref, ad2_ref, e8_ref,
             o16_ref, h2_ref, ast_ref, adt_ref, amx_ref):
        num = n_ref[0, : _N, :] + n_ref[1, : _N, :]
        den = d_ref[0, : _N, 0:8] + d_ref[1, : _N, 0:8]
        rec = 1.0 / (den + 1e-16)
        recb = jnp.dot(rec, e8_ref[...], preferred_element_type=jnp.float32)
        hin = num * recb + b1_ref[...]
        hin = jnp.where(hin > 0, hin, jnp.exp(jnp.minimum(hin, 0.0)) - 1.0)
        h2 = jnp.dot(hin, w2_ref[...], preferred_element_type=jnp.float32)
        h2_ref[...] = h2
        a_s = jnp.dot(h2, as2_ref[...], preferred_element_type=jnp.float32)
        a_d = jnp.dot(h2, ad2_ref[...], preferred_element_type=jnp.float32)
        ast_ref[...] = jnp.dot(a_s, o16_ref[...],
                               preferred_element_type=jnp.float32)
        adt_ref[...] = jnp.dot(a_d, o16_ref[...],
                               preferred_element_type=jnp.float32)
        m = jnp.max(a_s, axis=0, keepdims=True)
        amx_ref[...] = jnp.dot(m, o16_ref[...],
                               preferred_element_type=jnp.float32)

    return pl.pallas_call(
        body,
        out_shape=[
            jax.ShapeDtypeStruct((_N, _OUT), jnp.float32),
            jax.ShapeDtypeStruct((_N, 16), jnp.float32),
            jax.ShapeDtypeStruct((_N, 16), jnp.float32),
            jax.ShapeDtypeStruct((1, 16), jnp.float32),
        ],
    )(numP, denP, b1.reshape(1, -1), W2, a_src2.T, a_dst2.T, EXP8, ONES16)


def _tc3(numP, denP, b2, ONES64):
    def body(n_ref, d_ref, b2_ref, o64_ref, out_ref):
        num = n_ref[0, : _N, :] + n_ref[1, : _N, :]
        den = d_ref[0, : _N, 0:1] + d_ref[1, : _N, 0:1]
        rec = 1.0 / (den + 1e-16)
        recb = jnp.dot(rec, o64_ref[...], preferred_element_type=jnp.float32)
        out_ref[...] = num * recb + b2_ref[...]

    return pl.pallas_call(
        body,
        out_shape=jax.ShapeDtypeStruct((_N, _OUT), jnp.float32),
    )(numP, denP, b2.reshape(1, -1), ONES64)


def kernel(x, edge_index, W1, a_src1, a_dst1, b1, W2, a_src2, a_dst2, b2):
    loop = jnp.arange(_N, dtype=jnp.int32)
    pad = _NW * _EPWU - _ET
    src_u = jnp.concatenate(
        [edge_index[0], loop, jnp.zeros((pad,), jnp.int32)])
    dst_u = jnp.concatenate(
        [edge_index[1], loop, jnp.full((pad,), _N, jnp.int32)])
    slack = jnp.zeros((_NW, 256), jnp.int32)
    srcp = jnp.concatenate(
        [src_u.reshape(_NW, _EPWU), slack], axis=1).reshape(-1)
    dstp = jnp.concatenate(
        [dst_u.reshape(_NW, _EPWU), slack], axis=1).reshape(-1)

    eye8 = jnp.eye(_HEADS, dtype=jnp.float32)
    As1 = (a_src1[:, :, None] * eye8[:, None, :]).reshape(_IN, _HEADS)
    Ad1 = (a_dst1[:, :, None] * eye8[:, None, :]).reshape(_IN, _HEADS)
    EXP8 = jnp.repeat(eye8, _HID, axis=1)
    ONES16 = jnp.ones((1, 16), jnp.float32)
    ONES64 = jnp.ones((1, _OUT), jnp.float32)

    zN1 = jnp.zeros((_NP, _IN), jnp.float32)
    zN2 = jnp.zeros((_NP, _OUT), jnp.float32)
    zD = jnp.zeros((_NP, 16), jnp.float32)

    tpad = jnp.zeros((_NP - _N, 16), jnp.float32)

    h1, asT1, adT1, amx1 = _tc1(x, W1, As1, Ad1)
    num1, den1 = _sc_edge_kernel(_IN, _HEADS, 64)(
        srcp, dstp, h1, jnp.concatenate([asT1, tpad]),
        jnp.concatenate([adT1, tpad]), amx1.reshape(16), zN1, zD)

    h2, asT2, adT2, amx2 = _tc2(num1, den1, b1, W2, a_src2, a_dst2,
                                EXP8, ONES16)
    num2, den2 = _sc_edge_kernel(_OUT, 1, 128)(
        srcp, dstp, h2, jnp.concatenate([asT2, tpad]),
        jnp.concatenate([adT2, tpad]), amx2.reshape(16), zN2, zD)

    return _tc3(num2, den2, b2, ONES64)


# speedup vs baseline: 1.1678x; 1.0070x over previous
import functools

import jax
import jax.numpy as jnp
from jax import lax
from jax.experimental import pallas as pl
from jax.experimental.pallas import tpu as pltpu
from jax.experimental.pallas import tpu_sc as plsc

_N = 10000
_E = 320000
_IN = 128
_HID = 16
_HEADS = 8
_OUT = 64

_NP = 10240
_NW = 32
_ET = _E + _N
_EPWU = 10496
_EPWA = _EPWU + 256
_ETP = _EPWA * _NW
_RPT = _NP // 16


def _sc_edge_kernel(F, HEADS, C):
    NCH = _EPWU // C
    mesh = plsc.VectorSubcoreMesh(core_axis_name="c", subcore_axis_name="s")
    out_type = [
        jax.ShapeDtypeStruct((2, _NP, F), jnp.float32),
        jax.ShapeDtypeStruct((2, _NP, 16), jnp.float32),
    ]
    scratch = [
        pltpu.VMEM((2, C), jnp.int32),
        pltpu.VMEM((2, C), jnp.int32),
        pltpu.VMEM((2, C, 16), jnp.float32),
        pltpu.VMEM((2, C, 16), jnp.float32),
        pltpu.VMEM((C, 16), jnp.float32),
        pltpu.VMEM((2, C, F), jnp.float32),
        pltpu.VMEM((C, F), jnp.float32),
        pltpu.VMEM((16,), jnp.float32),
        pltpu.VMEM_SHARED((_NP, F), jnp.float32),
        pltpu.VMEM_SHARED((_NP, 16), jnp.float32),
        pltpu.SemaphoreType.DMA,
        pltpu.SemaphoreType.DMA,
        pltpu.SemaphoreType.DMA,
        pltpu.SemaphoreType.DMA,
        pltpu.SemaphoreType.DMA,
    ]

    @functools.partial(
        pl.kernel, mesh=mesh, out_type=out_type, scratch_types=scratch,
        compiler_params=pltpu.CompilerParams(use_tc_tiling_on_sc=False))
    def k(srcR, dstR, hT, asT, adT, amx, zN, zD, numP, denP,
          idxS, idxD, asg, adg, exv, hg, msg, amv, accN, accD,
          isem0, isem1, gsem0, gsem1, ssem):
        c = lax.axis_index("c")
        s = lax.axis_index("s")
        wid = s * 2 + c
        isem = [isem0, isem1]
        gsem = [gsem0, gsem1]

        @pl.when(s == 0)
        def _init():
            pltpu.sync_copy(zN, accN)
            pltpu.sync_copy(zD, accD)

        pltpu.sync_copy(amx, amv)
        plsc.subcore_barrier()
        amxv = amv[...]

        def idx_dma(g, b):
            base = wid * _EPWA + g * C
            return (pltpu.make_async_copy(srcR.at[pl.ds(base, C)],
                                          idxS.at[b], isem[b]),
                    pltpu.make_async_copy(dstR.at[pl.ds(base, C)],
                                          idxD.at[b], isem[b]))

        def gather_dma(b):
            return (pltpu.make_async_copy(asT.at[idxS.at[b]], asg.at[b],
                                          gsem[b]),
                    pltpu.make_async_copy(adT.at[idxD.at[b]], adg.at[b],
                                          gsem[b]),
                    pltpu.make_async_copy(hT.at[idxS.at[b]], hg.at[b],
                                          gsem[b]))

        def compute(b):
            @plsc.parallel_loop(0, C, 1, unroll=4)
            def edge_body(e):
                vs = asg[b, e, :]
                vd = adg[b, e, :]
                z = vs + vd
                ee = jnp.maximum(z, 0.2 * z)
                zm = amxv + vd
                mub = jnp.maximum(zm, 0.2 * zm)
                ex = jnp.exp(ee - mub)
                exv[e, :] = ex
                if HEADS == 1:
                    for q in range(F // 16):
                        msg[e, pl.ds(q * 16, 16)] = (
                            hg[b, e, pl.ds(q * 16, 16)] * ex)
                else:
                    for h in range(HEADS):
                        bb = lax.gather(
                            ex, jnp.full((16, 1), h, jnp.int32),
                            lax.GatherDimensionNumbers(
                                offset_dims=(), collapsed_slice_dims=(0,),
                                start_index_map=(0,)),
                            (1,),
                            mode=lax.GatherScatterMode.PROMISE_IN_BOUNDS)
                        msg[e, pl.ds(h * 16, 16)] = (
                            hg[b, e, pl.ds(h * 16, 16)] * bb)

        for d in idx_dma(0, 0):
            d.start()
        for d in idx_dma(1, 1):
            d.start()
        for d in idx_dma(0, 0):
            d.wait()
        for d in gather_dma(0):
            d.start()

        def loop_body(t, carry):
            for b in range(2):
                g = 2 * t + b
                nb = 1 - b
                for d in idx_dma(g + 1, nb):
                    d.wait()
                for d in gather_dma(nb):
                    d.start()
                for d in gather_dma(b):
                    d.wait()
                compute(b)
                d1 = pltpu.async_copy(exv, accD.at[idxD.at[b]],
                                      ssem, add=True)
                d2 = pltpu.async_copy(msg, accN.at[idxD.at[b]],
                                      ssem, add=True)
                d1.wait()
                d2.wait()
                for d in idx_dma(g + 2, b):
                    d.start()
            return carry

        lax.fori_loop(0, NCH // 2, loop_body, 0)

        for d in gather_dma(0):
            d.wait()
        for d in idx_dma(NCH + 1, 1):
            d.wait()

        plsc.subcore_barrier()
        r0 = s * _RPT
        pltpu.sync_copy(accN.at[pl.ds(r0, _RPT)],
                        numP.at[c, pl.ds(r0, _RPT)])
        pltpu.sync_copy(accD.at[pl.ds(r0, _RPT)],
                        denP.at[c, pl.ds(r0, _RPT)])

    return k


def _tc1(x, W1, As1, Ad1):
    def body(x_ref, w_ref, as_ref, ad_ref, h_ref, ast_ref, adt_ref, amx_ref):
        h = jnp.dot(x_ref[...], w_ref[...], preferred_element_type=jnp.float32)
        h_ref[...] = h
        a_s = jnp.dot(h, as_ref[...], preferred_element_type=jnp.float32)
        a_d = jnp.dot(h, ad_ref[...], preferred_element_type=jnp.float32)
        ast_ref[...] = jnp.concatenate([a_s, a_s], axis=1)
        adt_ref[...] = jnp.concatenate([a_d, a_d], axis=1)
        m = jnp.max(a_s, axis=0, keepdims=True)
        amx_ref[...] = jnp.concatenate([m, m], axis=1)

    return pl.pallas_call(
        body,
        out_shape=[
            jax.ShapeDtypeStruct((_N, _IN), jnp.float32),
            jax.ShapeDtypeStruct((_N, 16), jnp.float32),
            jax.ShapeDtypeStruct((_N, 16), jnp.float32),
            jax.ShapeDtypeStruct((1, 16), jnp.float32),
        ],
    )(x, W1, As1, Ad1)


def _tc2(numP, denP, b1, W2, a_src2, a_dst2, EXP8, ONES16):
    def body(n_ref, d_ref, b1_ref, w2_ref, as2_ref, ad2_ref, e8_ref,
             o16_ref, h2_ref, ast_ref, adt_ref, amx_ref):
        num = n_ref[0, : _N, :] + n_ref[1, : _N, :]
        den = d_ref[0, : _N, 0:8] + d_ref[1, : _N, 0:8]
        rec = 1.0 / (den + 1e-16)
        recb = jnp.dot(rec, e8_ref[...], preferred_element_type=jnp.float32)
        hin = num * recb + b1_ref[...]
        hin = jnp.where(hin > 0, hin, jnp.exp(jnp.minimum(hin, 0.0)) - 1.0)
        h2 = jnp.dot(hin, w2_ref[...], preferred_element_type=jnp.float32)
        h2_ref[...] = h2
        a_s = jnp.dot(h2, as2_ref[...], preferred_element_type=jnp.float32)
        a_d = jnp.dot(h2, ad2_ref[...], preferred_element_type=jnp.float32)
        ast_ref[...] = jnp.dot(a_s, o16_ref[...],
                               preferred_element_type=jnp.float32)
        adt_ref[...] = jnp.dot(a_d, o16_ref[...],
                               preferred_element_type=jnp.float32)
        m = jnp.max(a_s, axis=0, keepdims=True)
        amx_ref[...] = jnp.dot(m, o16_ref[...],
                               preferred_element_type=jnp.float32)

    return pl.pallas_call(
        body,
        out_shape=[
            jax.ShapeDtypeStruct((_N, _OUT), jnp.float32),
            jax.ShapeDtypeStruct((_N, 16), jnp.float32),
            jax.ShapeDtypeStruct((_N, 16), jnp.float32),
            jax.ShapeDtypeStruct((1, 16), jnp.float32),
        ],
    )(numP, denP, b1.reshape(1, -1), W2, a_src2.T, a_dst2.T, EXP8, ONES16)


def _tc3(numP, denP, b2, ONES64):
    def body(n_ref, d_ref, b2_ref, o64_ref, out_ref):
        num = n_ref[0, : _N, :] + n_ref[1, : _N, :]
        den = d_ref[0, : _N, 0:1] + d_ref[1, : _N, 0:1]
        rec = 1.0 / (den + 1e-16)
        recb = jnp.dot(rec, o64_ref[...], preferred_element_type=jnp.float32)
        out_ref[...] = num * recb + b2_ref[...]

    return pl.pallas_call(
        body,
        out_shape=jax.ShapeDtypeStruct((_N, _OUT), jnp.float32),
    )(numP, denP, b2.reshape(1, -1), ONES64)


def kernel(x, edge_index, W1, a_src1, a_dst1, b1, W2, a_src2, a_dst2, b2):
    loop = jnp.arange(_N, dtype=jnp.int32)
    pad = _NW * _EPWU - _ET
    src_u = jnp.concatenate(
        [edge_index[0], loop, jnp.zeros((pad,), jnp.int32)])
    dst_u = jnp.concatenate(
        [edge_index[1], loop, jnp.full((pad,), _N, jnp.int32)])
    slack = jnp.zeros((_NW, 256), jnp.int32)
    srcp = jnp.concatenate(
        [src_u.reshape(_NW, _EPWU), slack], axis=1).reshape(-1)
    dstp = jnp.concatenate(
        [dst_u.reshape(_NW, _EPWU), slack], axis=1).reshape(-1)

    eye8 = jnp.eye(_HEADS, dtype=jnp.float32)
    As1 = (a_src1[:, :, None] * eye8[:, None, :]).reshape(_IN, _HEADS)
    Ad1 = (a_dst1[:, :, None] * eye8[:, None, :]).reshape(_IN, _HEADS)
    EXP8 = jnp.repeat(eye8, _HID, axis=1)
    ONES16 = jnp.ones((1, 16), jnp.float32)
    ONES64 = jnp.ones((1, _OUT), jnp.float32)

    zN1 = jnp.zeros((_NP, _IN), jnp.float32)
    zN2 = jnp.zeros((_NP, _OUT), jnp.float32)
    zD = jnp.zeros((_NP, 16), jnp.float32)

    tpad = jnp.zeros((_NP - _N, 16), jnp.float32)

    h1, asT1, adT1, amx1 = _tc1(x, W1, As1, Ad1)
    num1, den1 = _sc_edge_kernel(_IN, _HEADS, 64)(
        srcp, dstp, h1, jnp.concatenate([asT1, tpad]),
        jnp.concatenate([adT1, tpad]), amx1.reshape(16), zN1, zD)

    h2, asT2, adT2, amx2 = _tc2(num1, den1, b1, W2, a_src2, a_dst2,
                                EXP8, ONES16)
    num2, den2 = _sc_edge_kernel(_OUT, 1, 128)(
        srcp, dstp, h2, jnp.concatenate([asT2, tpad]),
        jnp.concatenate([adT2, tpad]), amx2.reshape(16), zN2, zD)

    return _tc3(num2, den2, b2, ONES64)
